# baseline (speedup 1.0000x reference)
import functools

import jax
import jax.numpy as jnp
from jax import lax
from jax.experimental import pallas as pl
from jax.experimental.pallas import tpu as pltpu
from jax.experimental.pallas import tpu_sc as plsc

T = 4096
D = 1024
E = 8
K = 2
N = T * K
BM = 256
NB = N // BM + E
NPAD = NB * BM

NW = 32
_SC_MESH = dict(core_axis_name="c", subcore_axis_name="s")


def _routing_body(x_ref, wg_ref, bg_ref, idx_ref, w_ref):
    x = x_ref[...]
    logits = jnp.dot(x, wg_ref[...], preferred_element_type=jnp.float32)
    logits = logits + bg_ref[...]
    e_iota = lax.broadcasted_iota(jnp.int32, logits.shape, 1)
    m1 = jnp.max(logits, axis=1, keepdims=True)
    i1 = jnp.min(jnp.where(logits == m1, e_iota, E), axis=1, keepdims=True)
    masked = jnp.where(e_iota == i1, -jnp.inf, logits)
    m2 = jnp.max(masked, axis=1, keepdims=True)
    i2 = jnp.min(jnp.where(masked == m2, e_iota, E), axis=1, keepdims=True)
    z = jnp.exp(m2 - m1)
    w1 = 1.0 / (1.0 + z)
    idx_ref[...] = jnp.concatenate([i1, i2], axis=1)
    w_ref[...] = jnp.concatenate([w1, 1.0 - w1], axis=1)


def _routing(inputs, Wg, bg):
    bt = 2048
    return pl.pallas_call(
        _routing_body,
        grid=(T // bt,),
        in_specs=[
            pl.BlockSpec((bt, D), lambda i: (i, 0)),
            pl.BlockSpec((D, E), lambda i: (0, 0)),
            pl.BlockSpec((1, E), lambda i: (0, 0)),
        ],
        out_specs=[
            pl.BlockSpec((bt, K), lambda i: (i, 0)),
            pl.BlockSpec((bt, K), lambda i: (i, 0)),
        ],
        out_shape=[
            jax.ShapeDtypeStruct((T, K), jnp.int32),
            jax.ShapeDtypeStruct((T, K), jnp.float32),
        ],
    )(inputs, Wg, bg.reshape(1, E))


def _dispatch_body(x_hbm, src_hbm, out_hbm, idx_v, rows_v, sem):
    ch = idx_v.shape[0]
    per_w = NPAD // NW
    wid = lax.axis_index("s") * 2 + lax.axis_index("c")

    def chunk(c, _):
        base = wid * per_w + c * ch
        pltpu.sync_copy(src_hbm.at[pl.ds(base, ch)], idx_v)
        pltpu.async_copy(x_hbm.at[idx_v], rows_v, sem).wait()
        pltpu.sync_copy(rows_v, out_hbm.at[pl.ds(base, ch)])
        return 0

    lax.fori_loop(0, per_w // ch, chunk, 0)


def _dispatch(inputs, src_token):
    ch = 64
    return pl.kernel(
        _dispatch_body,
        out_type=jax.ShapeDtypeStruct((NPAD, D), jnp.float32),
        mesh=plsc.VectorSubcoreMesh(**_SC_MESH),
        scratch_types=[
            pltpu.VMEM((ch,), jnp.int32),
            pltpu.VMEM((ch, D), jnp.float32),
            pltpu.SemaphoreType.DMA,
        ],
    )(inputs, src_token)


def _gmm_body(bmap_ref, x_ref, we_ref, be_ref, ws_ref, y_ref):
    del bmap_ref
    y = jnp.dot(x_ref[...], we_ref[0], preferred_element_type=jnp.float32)
    y_ref[...] = (y + be_ref[0]) * ws_ref[...]


def _gmm(x_pad, We, be, ws, block_expert):
    grid_spec = pltpu.PrefetchScalarGridSpec(
        num_scalar_prefetch=1,
        grid=(NB,),
        in_specs=[
            pl.BlockSpec((BM, D), lambda i, bmap: (i, 0)),
            pl.BlockSpec((1, D, D), lambda i, bmap: (bmap[i], 0, 0)),
            pl.BlockSpec((1, 1, D), lambda i, bmap: (bmap[i], 0, 0)),
            pl.BlockSpec((BM, 1), lambda i, bmap: (i, 0)),
        ],
        out_specs=pl.BlockSpec((BM, D), lambda i, bmap: (i, 0)),
    )
    return pl.pallas_call(
        _gmm_body,
        grid_spec=grid_spec,
        out_shape=jax.ShapeDtypeStruct((NPAD, D), jnp.float32),
    )(block_expert, x_pad, We, be.reshape(E, 1, D), ws.reshape(NPAD, 1))


def _combine_body(y_hbm, p0_hbm, p1_hbm, out_hbm, i0_v, i1_v, r0_v, r1_v, sem):
    ct = i0_v.shape[0]
    per_w = T // NW
    wid = lax.axis_index("s") * 2 + lax.axis_index("c")

    def chunk(c, _):
        base = wid * per_w + c * ct
        pltpu.sync_copy(p0_hbm.at[pl.ds(base, ct)], i0_v)
        pltpu.sync_copy(p1_hbm.at[pl.ds(base, ct)], i1_v)
        pltpu.async_copy(y_hbm.at[i0_v], r0_v, sem).wait()
        pltpu.async_copy(y_hbm.at[i1_v], r1_v, sem).wait()

        def add_row(j, _):
            def add_slice(k, _):
                s = pl.ds(k * 16, 16)
                r0_v[j, s] = r0_v[j, s] + r1_v[j, s]
                return 0
            lax.fori_loop(0, D // 16, add_slice, 0)
            return 0

        lax.fori_loop(0, ct, add_row, 0)
        pltpu.sync_copy(r0_v, out_hbm.at[pl.ds(base, ct)])
        return 0

    lax.fori_loop(0, per_w // ct, chunk, 0)


def _combine(y_pad, pos0, pos1):
    ct = 32
    return pl.kernel(
        _combine_body,
        out_type=jax.ShapeDtypeStruct((T, D), jnp.float32),
        mesh=plsc.VectorSubcoreMesh(**_SC_MESH),
        scratch_types=[
            pltpu.VMEM((ct,), jnp.int32),
            pltpu.VMEM((ct,), jnp.int32),
            pltpu.VMEM((ct, D), jnp.float32),
            pltpu.VMEM((ct, D), jnp.float32),
            pltpu.SemaphoreType.DMA,
        ],
    )(y_pad, pos0, pos1)


def _schedule(top_idx, w):
    e_flat = top_idx.reshape(-1)
    oh = (e_flat[:, None] == jnp.arange(E)[None, :]).astype(jnp.int32)
    counts = oh.sum(axis=0)
    blocks_per = (counts + BM - 1) // BM
    cumblocks = jnp.cumsum(blocks_per)
    padded_start = (cumblocks - blocks_per) * BM
    rank = jnp.cumsum(oh, axis=0) - oh
    rank_p = jnp.take_along_axis(rank, e_flat[:, None], axis=1)[:, 0]
    dest = padded_start[e_flat] + rank_p
    src_token = jnp.zeros((NPAD,), jnp.int32).at[dest].set(
        jnp.arange(N, dtype=jnp.int32) // K)
    ws = jnp.zeros((NPAD,), jnp.float32).at[dest].set(w.reshape(-1))
    block_expert = jnp.searchsorted(
        cumblocks, jnp.arange(NB, dtype=jnp.int32), side="right")
    block_expert = jnp.minimum(block_expert, E - 1).astype(jnp.int32)
    pos = dest.reshape(T, K)
    return src_token, ws, block_expert, pos[:, 0], pos[:, 1]


def kernel(inputs, Wg, bg, We, be):
    top_idx, w = _routing(inputs, Wg, bg)
    src_token, ws, block_expert, pos0, pos1 = _schedule(top_idx, w)
    x_pad = _dispatch(inputs, src_token)
    y_pad = _gmm(x_pad, We, be, ws, block_expert)
    return _combine(y_pad, pos0, pos1)


# speedup vs baseline: 1.7154x; 1.7154x over previous
import functools

import jax
import jax.numpy as jnp
from jax import lax
from jax.experimental import pallas as pl
from jax.experimental.pallas import tpu as pltpu
from jax.experimental.pallas import tpu_sc as plsc

T = 4096
D = 1024
E = 8
K = 2
N = T * K
BM = 256
NB = N // BM + E
NPAD = NB * BM

NW = 32
_SC_MESH = dict(core_axis_name="c", subcore_axis_name="s")


def _routing_body(x_ref, wg_ref, bg_ref, idx_ref, w_ref):
    x = x_ref[...]
    logits = jnp.dot(x, wg_ref[...], preferred_element_type=jnp.float32)
    logits = logits + bg_ref[...]
    e_iota = lax.broadcasted_iota(jnp.int32, logits.shape, 1)
    m1 = jnp.max(logits, axis=1, keepdims=True)
    i1 = jnp.min(jnp.where(logits == m1, e_iota, E), axis=1, keepdims=True)
    masked = jnp.where(e_iota == i1, -jnp.inf, logits)
    m2 = jnp.max(masked, axis=1, keepdims=True)
    i2 = jnp.min(jnp.where(masked == m2, e_iota, E), axis=1, keepdims=True)
    z = jnp.exp(m2 - m1)
    w1 = 1.0 / (1.0 + z)
    idx_ref[...] = jnp.concatenate([i1, i2], axis=1)
    w_ref[...] = jnp.concatenate([w1, 1.0 - w1], axis=1)


def _routing(inputs, Wg, bg):
    bt = 2048
    return pl.pallas_call(
        _routing_body,
        grid=(T // bt,),
        in_specs=[
            pl.BlockSpec((bt, D), lambda i: (i, 0)),
            pl.BlockSpec((D, E), lambda i: (0, 0)),
            pl.BlockSpec((1, E), lambda i: (0, 0)),
        ],
        out_specs=[
            pl.BlockSpec((bt, K), lambda i: (i, 0)),
            pl.BlockSpec((bt, K), lambda i: (i, 0)),
        ],
        out_shape=[
            jax.ShapeDtypeStruct((T, K), jnp.int32),
            jax.ShapeDtypeStruct((T, K), jnp.float32),
        ],
    )(inputs, Wg, bg.reshape(1, E))


_DCT = 32
_DNCH = (T // NW) // _DCT


def _dispatch_body(x_hbm, p0_hbm, p1_hbm, out_hbm, i0_v, i1_v, xin_v,
                   lsem, ssem):
    per_w = T // NW
    wid = lax.axis_index("s") * 2 + lax.axis_index("c")
    tbase = wid * per_w

    def start_load(c):
        b = c % 2
        base = tbase + c * _DCT
        return (
            pltpu.async_copy(p0_hbm.at[pl.ds(base, _DCT)], i0_v.at[b], lsem),
            pltpu.async_copy(p1_hbm.at[pl.ds(base, _DCT)], i1_v.at[b], lsem),
            pltpu.async_copy(x_hbm.at[pl.ds(base, _DCT)], xin_v.at[b], lsem),
        )

    loads = {0: start_load(0), 1: start_load(1)}
    scats = {}
    for c in range(_DNCH):
        b = c % 2
        for h in loads[c]:
            h.wait()
        scats[c] = (
            pltpu.async_copy(xin_v.at[b], out_hbm.at[i0_v.at[b]], ssem),
            pltpu.async_copy(xin_v.at[b], out_hbm.at[i1_v.at[b]], ssem),
        )
        if c + 2 < _DNCH:
            for h in scats[c]:
                h.wait()
            loads[c + 2] = start_load(c + 2)
    for c in (_DNCH - 2, _DNCH - 1):
        for h in scats[c]:
            h.wait()


def _dispatch(inputs, pos0, pos1):
    return pl.kernel(
        _dispatch_body,
        out_type=jax.ShapeDtypeStruct((NPAD, D), jnp.float32),
        mesh=plsc.VectorSubcoreMesh(**_SC_MESH),
        scratch_types=[
            pltpu.VMEM((2, _DCT), jnp.int32),
            pltpu.VMEM((2, _DCT), jnp.int32),
            pltpu.VMEM((2, _DCT, D), jnp.float32),
            pltpu.SemaphoreType.DMA,
            pltpu.SemaphoreType.DMA,
        ],
    )(inputs, pos0, pos1)


def _gmm_body(bmap_ref, bval_ref, x_ref, we_ref, be_ref, ws_ref, y_ref):
    del bmap_ref

    @pl.when(bval_ref[pl.program_id(0)] != 0)
    def _():
        y = jnp.dot(x_ref[...], we_ref[0], preferred_element_type=jnp.float32)
        y_ref[...] = (y + be_ref[0]) * ws_ref[...]


def _gmm(x_pad, We, be, ws, block_expert, block_valid):
    grid_spec = pltpu.PrefetchScalarGridSpec(
        num_scalar_prefetch=2,
        grid=(NB,),
        in_specs=[
            pl.BlockSpec((BM, D), lambda i, bmap, bval: (i, 0)),
            pl.BlockSpec((1, D, D), lambda i, bmap, bval: (bmap[i], 0, 0)),
            pl.BlockSpec((1, 1, D), lambda i, bmap, bval: (bmap[i], 0, 0)),
            pl.BlockSpec((BM, 1), lambda i, bmap, bval: (i, 0)),
        ],
        out_specs=pl.BlockSpec((BM, D), lambda i, bmap, bval: (i, 0)),
    )
    return pl.pallas_call(
        _gmm_body,
        grid_spec=grid_spec,
        out_shape=jax.ShapeDtypeStruct((NPAD, D), jnp.float32),
    )(block_expert, block_valid, x_pad, We, be.reshape(E, 1, D),
      ws.reshape(NPAD, 1))


_CCT = 16
_CNCH = (T // NW) // _CCT


def _combine_body(y_hbm, p0_hbm, p1_hbm, out_hbm, i0_v, i1_v, r0_v, r1_v, o_v,
                  isem, gsem, wsem):
    per_w = T // NW
    wid = lax.axis_index("s") * 2 + lax.axis_index("c")
    tbase = wid * per_w

    pltpu.async_copy(p0_hbm.at[pl.ds(tbase, per_w)], i0_v, isem).wait()
    pltpu.async_copy(p1_hbm.at[pl.ds(tbase, per_w)], i1_v, isem).wait()

    def start_gather(c):
        b = c % 2
        s = pl.ds(c * _CCT, _CCT)
        return (
            pltpu.async_copy(y_hbm.at[i0_v.at[s]], r0_v.at[b], gsem),
            pltpu.async_copy(y_hbm.at[i1_v.at[s]], r1_v.at[b], gsem),
        )

    gath = {0: start_gather(0), 1: start_gather(1)}
    stores = {}
    for c in range(_CNCH):
        b = c % 2
        for h in gath[c]:
            h.wait()
        if c >= 2:
            stores[c - 2].wait()

        def add_row(j, _):
            def add4(k, _):
                for u in range(4):
                    s = pl.ds((k * 4 + u) * 16, 16)
                    o_v[b, j, s] = r0_v[b, j, s] + r1_v[b, j, s]
                return 0
            lax.fori_loop(0, D // 64, add4, 0)
            return 0

        lax.fori_loop(0, _CCT, add_row, 0)
        stores[c] = pltpu.async_copy(
            o_v.at[b], out_hbm.at[pl.ds(tbase + c * _CCT, _CCT)], wsem)
        if c + 2 < _CNCH:
            gath[c + 2] = start_gather(c + 2)
    stores[_CNCH - 2].wait()
    stores[_CNCH - 1].wait()


def _combine(y_pad, pos0, pos1):
    per_w = T // NW
    return pl.kernel(
        _combine_body,
        out_type=jax.ShapeDtypeStruct((T, D), jnp.float32),
        mesh=plsc.VectorSubcoreMesh(**_SC_MESH),
        scratch_types=[
            pltpu.VMEM((per_w,), jnp.int32),
            pltpu.VMEM((per_w,), jnp.int32),
            pltpu.VMEM((2, _CCT, D), jnp.float32),
            pltpu.VMEM((2, _CCT, D), jnp.float32),
            pltpu.VMEM((2, _CCT, D), jnp.float32),
            pltpu.SemaphoreType.DMA,
            pltpu.SemaphoreType.DMA,
            pltpu.SemaphoreType.DMA,
        ],
    )(y_pad, pos0, pos1)


def _schedule(top_idx, w):
    e_flat = top_idx.reshape(-1)
    oh = (e_flat[:, None] == jnp.arange(E)[None, :]).astype(jnp.int32)
    counts = oh.sum(axis=0)
    blocks_per = (counts + BM - 1) // BM
    cumblocks = jnp.cumsum(blocks_per)
    padded_start = (cumblocks - blocks_per) * BM
    rank = jnp.cumsum(oh, axis=0) - oh
    rank_p = jnp.take_along_axis(rank, e_flat[:, None], axis=1)[:, 0]
    dest = padded_start[e_flat] + rank_p
    ws = jnp.zeros((NPAD,), jnp.float32).at[dest].set(w.reshape(-1))
    barange = jnp.arange(NB, dtype=jnp.int32)
    block_expert = jnp.searchsorted(cumblocks, barange, side="right")
    block_expert = jnp.minimum(block_expert, E - 1).astype(jnp.int32)
    block_valid = (barange < cumblocks[-1]).astype(jnp.int32)
    pos = dest.reshape(T, K)
    return ws, block_expert, block_valid, pos[:, 0], pos[:, 1]


def kernel(inputs, Wg, bg, We, be):
    top_idx, w = _routing(inputs, Wg, bg)
    ws, block_expert, block_valid, pos0, pos1 = _schedule(top_idx, w)
    x_pad = _dispatch(inputs, pos0, pos1)
    y_pad = _gmm(x_pad, We, be, ws, block_expert, block_valid)
    return _combine(y_pad, pos0, pos1)


# speedup vs baseline: 2.4721x; 1.4411x over previous
import functools

import jax
import jax.numpy as jnp
from jax import lax
from jax.experimental import pallas as pl
from jax.experimental.pallas import tpu as pltpu
from jax.experimental.pallas import tpu_sc as plsc

T = 4096
D = 1024
E = 8
K = 2
N = T * K
BM = 256
NB = N // BM + E
NPAD = NB * BM

NW = 32
_SC_MESH = dict(core_axis_name="c", subcore_axis_name="s")


def _excl_cumsum(x):
    z = jnp.zeros_like(x)
    acc = x
    sh = 1
    while sh < T:
        acc = acc + jnp.concatenate([z[:sh], acc[: T - sh]], axis=0)
        sh *= 2
    return acc - x


def _route_sched_body(x_ref, wg_ref, bg_ref,
                      p0_ref, p1_ref, w0_ref, w1_ref, bmap_ref, bval_ref):
    x = x_ref[...]
    logits = jnp.dot(x, wg_ref[...], preferred_element_type=jnp.float32)
    logits = logits + bg_ref[...]
    e_iota = lax.broadcasted_iota(jnp.int32, logits.shape, 1)
    m1 = jnp.max(logits, axis=1, keepdims=True)
    i1 = jnp.min(jnp.where(logits == m1, e_iota, E), axis=1, keepdims=True)
    masked = jnp.where(e_iota == i1, -jnp.inf, logits)
    m2 = jnp.max(masked, axis=1, keepdims=True)
    i2 = jnp.min(jnp.where(masked == m2, e_iota, E), axis=1, keepdims=True)
    z = jnp.exp(m2 - m1)
    w1 = 1.0 / (1.0 + z)
    w0_ref[...] = jnp.broadcast_to(w1, (T, 16))
    w1_ref[...] = jnp.broadcast_to(1.0 - w1, (T, 16))

    oh1 = (e_iota == i1).astype(jnp.int32)
    oh2 = (e_iota == i2).astype(jnp.int32)
    r1 = _excl_cumsum(oh1)
    r2 = _excl_cumsum(oh2)
    c1 = r1[T - 1:T, :] + oh1[T - 1:T, :]
    c2 = r2[T - 1:T, :] + oh2[T - 1:T, :]
    counts = c1 + c2
    blocks_per = (counts + BM - 1) // BM
    ri = lax.broadcasted_iota(jnp.int32, (E, E), 0)
    ci = lax.broadcasted_iota(jnp.int32, (E, E), 1)
    tri = (ri <= ci).astype(jnp.float32)
    cumblocks = jnp.dot(blocks_per.astype(jnp.float32), tri,
                        preferred_element_type=jnp.float32).astype(jnp.int32)
    padded_start = (cumblocks - blocks_per) * BM

    dest1 = jnp.sum(oh1 * (padded_start + r1), axis=1, keepdims=True)
    dest2 = jnp.sum(oh2 * (padded_start + c1 + r2), axis=1, keepdims=True)
    p0_ref[...] = dest1
    p1_ref[...] = dest2

    b_iota = lax.broadcasted_iota(jnp.int32, (NB, E), 0)
    bmap = jnp.sum((jnp.broadcast_to(cumblocks, (NB, E)) <= b_iota)
                   .astype(jnp.int32), axis=1, keepdims=True)
    bmap_ref[...] = jnp.minimum(bmap, E - 1)
    bval_ref[...] = (b_iota[:, :1] < cumblocks[0, E - 1]).astype(jnp.int32)


def _routing(inputs, Wg, bg):
    return pl.pallas_call(
        _route_sched_body,
        grid=(1,),
        in_specs=[
            pl.BlockSpec((T, D), lambda i: (0, 0)),
            pl.BlockSpec((D, E), lambda i: (0, 0)),
            pl.BlockSpec((1, E), lambda i: (0, 0)),
        ],
        out_specs=[
            pl.BlockSpec((T, 1), lambda i: (0, 0)),
            pl.BlockSpec((T, 1), lambda i: (0, 0)),
            pl.BlockSpec((T, 16), lambda i: (0, 0)),
            pl.BlockSpec((T, 16), lambda i: (0, 0)),
            pl.BlockSpec((NB, 1), lambda i: (0, 0)),
            pl.BlockSpec((NB, 1), lambda i: (0, 0)),
        ],
        out_shape=[
            jax.ShapeDtypeStruct((T, 1), jnp.int32),
            jax.ShapeDtypeStruct((T, 1), jnp.int32),
            jax.ShapeDtypeStruct((T, 16), jnp.float32),
            jax.ShapeDtypeStruct((T, 16), jnp.float32),
            jax.ShapeDtypeStruct((NB, 1), jnp.int32),
            jax.ShapeDtypeStruct((NB, 1), jnp.int32),
        ],
    )(inputs, Wg, bg.reshape(1, E))


_DCT = 32
_DNCH = (T // NW) // _DCT


def _dispatch_body(x_hbm, p0_hbm, p1_hbm, out_hbm, i0_v, i1_v, xin_v,
                   lsem, ssem):
    per_w = T // NW
    wid = lax.axis_index("s") * 2 + lax.axis_index("c")
    tbase = wid * per_w

    def start_load(c):
        b = c % 2
        base = tbase + c * _DCT
        return (
            pltpu.async_copy(p0_hbm.at[pl.ds(base, _DCT)], i0_v.at[b], lsem),
            pltpu.async_copy(p1_hbm.at[pl.ds(base, _DCT)], i1_v.at[b], lsem),
            pltpu.async_copy(x_hbm.at[pl.ds(base, _DCT)], xin_v.at[b], lsem),
        )

    loads = {0: start_load(0), 1: start_load(1)}
    scats = {}
    for c in range(_DNCH):
        b = c % 2
        for h in loads[c]:
            h.wait()
        scats[c] = (
            pltpu.async_copy(xin_v.at[b], out_hbm.at[i0_v.at[b]], ssem),
            pltpu.async_copy(xin_v.at[b], out_hbm.at[i1_v.at[b]], ssem),
        )
        if c + 2 < _DNCH:
            for h in scats[c]:
                h.wait()
            loads[c + 2] = start_load(c + 2)
    for c in (_DNCH - 2, _DNCH - 1):
        for h in scats[c]:
            h.wait()


def _dispatch(inputs, pos0, pos1):
    return pl.kernel(
        _dispatch_body,
        out_type=jax.ShapeDtypeStruct((NPAD, D), jnp.float32),
        mesh=plsc.VectorSubcoreMesh(**_SC_MESH),
        scratch_types=[
            pltpu.VMEM((2, _DCT), jnp.int32),
            pltpu.VMEM((2, _DCT), jnp.int32),
            pltpu.VMEM((2, _DCT, D), jnp.float32),
            pltpu.SemaphoreType.DMA,
            pltpu.SemaphoreType.DMA,
        ],
    )(inputs, pos0, pos1)


def _gmm_body(bmap_ref, bval_ref, x_ref, we_ref, be_ref, y_ref):
    del bmap_ref

    @pl.when(bval_ref[pl.program_id(0)] != 0)
    def _():
        y = jnp.dot(x_ref[...], we_ref[0], preferred_element_type=jnp.float32)
        y_ref[...] = y + be_ref[0]


def _gmm(x_pad, We, be, block_expert, block_valid):
    grid_spec = pltpu.PrefetchScalarGridSpec(
        num_scalar_prefetch=2,
        grid=(NB,),
        in_specs=[
            pl.BlockSpec((BM, D), lambda i, bmap, bval: (i, 0)),
            pl.BlockSpec((1, D, D), lambda i, bmap, bval: (bmap[i], 0, 0)),
            pl.BlockSpec((1, 1, D), lambda i, bmap, bval: (bmap[i], 0, 0)),
        ],
        out_specs=pl.BlockSpec((BM, D), lambda i, bmap, bval: (i, 0)),
    )
    return pl.pallas_call(
        _gmm_body,
        grid_spec=grid_spec,
        out_shape=jax.ShapeDtypeStruct((NPAD, D), jnp.float32),
    )(block_expert, block_valid, x_pad, We, be.reshape(E, 1, D))


_CCT = 16
_CNCH = (T // NW) // _CCT


def _combine_body(y_hbm, p0_hbm, p1_hbm, w0_hbm, w1_hbm, out_hbm,
                  i0_v, i1_v, w0_v, w1_v, r0_v, r1_v, o_v,
                  isem, gsem, wsem):
    per_w = T // NW
    wid = lax.axis_index("s") * 2 + lax.axis_index("c")
    tbase = wid * per_w

    pltpu.async_copy(p0_hbm.at[pl.ds(tbase, per_w)], i0_v, isem).wait()
    pltpu.async_copy(p1_hbm.at[pl.ds(tbase, per_w)], i1_v, isem).wait()

    def start_gather(c):
        b = c % 2
        s = pl.ds(c * _CCT, _CCT)
        hs = pl.ds(tbase + c * _CCT, _CCT)
        return (
            pltpu.async_copy(y_hbm.at[i0_v.at[s]], r0_v.at[b], gsem),
            pltpu.async_copy(y_hbm.at[i1_v.at[s]], r1_v.at[b], gsem),
            pltpu.async_copy(w0_hbm.at[hs], w0_v.at[b], gsem),
            pltpu.async_copy(w1_hbm.at[hs], w1_v.at[b], gsem),
        )

    gath = {0: start_gather(0), 1: start_gather(1)}
    stores = {}
    for c in range(_CNCH):
        b = c % 2
        for h in gath[c]:
            h.wait()
        if c >= 2:
            stores[c - 2].wait()

        def add_row(j, _):
            wa = w0_v[b, j, :]
            wb = w1_v[b, j, :]

            def add4(k, _):
                for u in range(4):
                    s = pl.ds((k * 4 + u) * 16, 16)
                    o_v[b, j, s] = r0_v[b, j, s] * wa + r1_v[b, j, s] * wb
                return 0
            lax.fori_loop(0, D // 64, add4, 0)
            return 0

        lax.fori_loop(0, _CCT, add_row, 0)
        stores[c] = pltpu.async_copy(
            o_v.at[b], out_hbm.at[pl.ds(tbase + c * _CCT, _CCT)], wsem)
        if c + 2 < _CNCH:
            gath[c + 2] = start_gather(c + 2)
    stores[_CNCH - 2].wait()
    stores[_CNCH - 1].wait()


def _combine(y_pad, pos0, pos1, w0e, w1e):
    per_w = T // NW
    return pl.kernel(
        _combine_body,
        out_type=jax.ShapeDtypeStruct((T, D), jnp.float32),
        mesh=plsc.VectorSubcoreMesh(**_SC_MESH),
        scratch_types=[
            pltpu.VMEM((per_w,), jnp.int32),
            pltpu.VMEM((per_w,), jnp.int32),
            pltpu.VMEM((2, _CCT, 16), jnp.float32),
            pltpu.VMEM((2, _CCT, 16), jnp.float32),
            pltpu.VMEM((2, _CCT, D), jnp.float32),
            pltpu.VMEM((2, _CCT, D), jnp.float32),
            pltpu.VMEM((2, _CCT, D), jnp.float32),
            pltpu.SemaphoreType.DMA,
            pltpu.SemaphoreType.DMA,
            pltpu.SemaphoreType.DMA,
        ],
    )(y_pad, pos0, pos1, w0e, w1e)


def kernel(inputs, Wg, bg, We, be):
    p0, p1, w0e, w1e, bmap, bval = _routing(inputs, Wg, bg)
    pos0 = p0.reshape(T)
    pos1 = p1.reshape(T)
    x_pad = _dispatch(inputs, pos0, pos1)
    y_pad = _gmm(x_pad, We, be, bmap.reshape(NB), bval.reshape(NB))
    return _combine(y_pad, pos0, pos1, w0e, w1e)


# speedup vs baseline: 2.4822x; 1.0041x over previous
import functools

import jax
import jax.numpy as jnp
from jax import lax
from jax.experimental import pallas as pl
from jax.experimental.pallas import tpu as pltpu
from jax.experimental.pallas import tpu_sc as plsc

T = 4096
D = 1024
E = 8
K = 2
N = T * K
BM = 256
NB = N // BM + E
NPAD = NB * BM

NW = 32
_SC_MESH = dict(core_axis_name="c", subcore_axis_name="s")


def _excl_cumsum(x):
    z = jnp.zeros_like(x)
    acc = x
    sh = 1
    while sh < T:
        acc = acc + jnp.concatenate([z[:sh], acc[: T - sh]], axis=0)
        sh *= 2
    return acc - x


def _route_sched_body(x_ref, wg_ref, bg_ref,
                      p0_ref, p1_ref, w0_ref, w1_ref, bmap_ref, bval_ref):
    x = x_ref[...]
    logits = jnp.dot(x, wg_ref[...], preferred_element_type=jnp.float32)
    logits = logits + bg_ref[...]
    e_iota = lax.broadcasted_iota(jnp.int32, logits.shape, 1)
    m1 = jnp.max(logits, axis=1, keepdims=True)
    i1 = jnp.min(jnp.where(logits == m1, e_iota, E), axis=1, keepdims=True)
    masked = jnp.where(e_iota == i1, -jnp.inf, logits)
    m2 = jnp.max(masked, axis=1, keepdims=True)
    i2 = jnp.min(jnp.where(masked == m2, e_iota, E), axis=1, keepdims=True)
    z = jnp.exp(m2 - m1)
    w1 = 1.0 / (1.0 + z)
    w0_ref[...] = jnp.broadcast_to(w1, (T, 16))
    w1_ref[...] = jnp.broadcast_to(1.0 - w1, (T, 16))

    oh1 = (e_iota == i1).astype(jnp.int32)
    oh2 = (e_iota == i2).astype(jnp.int32)
    oh = oh1 + oh2
    r = _excl_cumsum(oh)
    counts = r[T - 1:T, :] + oh[T - 1:T, :]
    blocks_per = (counts + BM - 1) // BM
    ri = lax.broadcasted_iota(jnp.int32, (E, E), 0)
    ci = lax.broadcasted_iota(jnp.int32, (E, E), 1)
    tri = (ri <= ci).astype(jnp.float32)
    cumblocks = jnp.dot(blocks_per.astype(jnp.float32), tri,
                        preferred_element_type=jnp.float32).astype(jnp.int32)
    padded_start = (cumblocks - blocks_per) * BM

    dest1 = jnp.sum(oh1 * (padded_start + r), axis=1, keepdims=True)
    dest2 = jnp.sum(oh2 * (padded_start + r + oh1), axis=1, keepdims=True)
    p0_ref[...] = dest1
    p1_ref[...] = dest2

    b_iota = lax.broadcasted_iota(jnp.int32, (NB, E), 0)
    bmap = jnp.sum((jnp.broadcast_to(cumblocks, (NB, E)) <= b_iota)
                   .astype(jnp.int32), axis=1, keepdims=True)
    bmap_ref[...] = jnp.minimum(bmap, E - 1)
    bval_ref[...] = (b_iota[:, :1] < cumblocks[0, E - 1]).astype(jnp.int32)


def _routing(inputs, Wg, bg):
    return pl.pallas_call(
        _route_sched_body,
        grid=(1,),
        in_specs=[
            pl.BlockSpec((T, D), lambda i: (0, 0)),
            pl.BlockSpec((D, E), lambda i: (0, 0)),
            pl.BlockSpec((1, E), lambda i: (0, 0)),
        ],
        out_specs=[
            pl.BlockSpec((T, 1), lambda i: (0, 0)),
            pl.BlockSpec((T, 1), lambda i: (0, 0)),
            pl.BlockSpec((T, 16), lambda i: (0, 0)),
            pl.BlockSpec((T, 16), lambda i: (0, 0)),
            pl.BlockSpec((NB, 1), lambda i: (0, 0)),
            pl.BlockSpec((NB, 1), lambda i: (0, 0)),
        ],
        out_shape=[
            jax.ShapeDtypeStruct((T, 1), jnp.int32),
            jax.ShapeDtypeStruct((T, 1), jnp.int32),
            jax.ShapeDtypeStruct((T, 16), jnp.float32),
            jax.ShapeDtypeStruct((T, 16), jnp.float32),
            jax.ShapeDtypeStruct((NB, 1), jnp.int32),
            jax.ShapeDtypeStruct((NB, 1), jnp.int32),
        ],
    )(inputs, Wg, bg.reshape(1, E))


_DCT = 32
_DNCH = (T // NW) // _DCT


def _dispatch_body(x_hbm, p0_hbm, p1_hbm, out_hbm, i0_v, i1_v, xin_v,
                   lsem, ssem):
    per_w = T // NW
    wid = lax.axis_index("s") * 2 + lax.axis_index("c")
    tbase = wid * per_w

    def start_load(c):
        b = c % 2
        base = tbase + c * _DCT
        return (
            pltpu.async_copy(p0_hbm.at[pl.ds(base, _DCT)], i0_v.at[b], lsem),
            pltpu.async_copy(p1_hbm.at[pl.ds(base, _DCT)], i1_v.at[b], lsem),
            pltpu.async_copy(x_hbm.at[pl.ds(base, _DCT)], xin_v.at[b], lsem),
        )

    loads = {0: start_load(0), 1: start_load(1)}
    scats = {}
    for c in range(_DNCH):
        b = c % 2
        for h in loads[c]:
            h.wait()
        scats[c] = (
            pltpu.async_copy(xin_v.at[b], out_hbm.at[i0_v.at[b]], ssem),
            pltpu.async_copy(xin_v.at[b], out_hbm.at[i1_v.at[b]], ssem),
        )
        if c + 2 < _DNCH:
            for h in scats[c]:
                h.wait()
            loads[c + 2] = start_load(c + 2)
    for c in (_DNCH - 2, _DNCH - 1):
        for h in scats[c]:
            h.wait()


def _dispatch(inputs, pos0, pos1):
    return pl.kernel(
        _dispatch_body,
        out_type=jax.ShapeDtypeStruct((NPAD, D), jnp.float32),
        mesh=plsc.VectorSubcoreMesh(**_SC_MESH),
        scratch_types=[
            pltpu.VMEM((2, _DCT), jnp.int32),
            pltpu.VMEM((2, _DCT), jnp.int32),
            pltpu.VMEM((2, _DCT, D), jnp.float32),
            pltpu.SemaphoreType.DMA,
            pltpu.SemaphoreType.DMA,
        ],
    )(inputs, pos0, pos1)


def _gmm_body(bmap_ref, bval_ref, x_ref, we_ref, be_ref, y_ref):
    del bmap_ref

    @pl.when(bval_ref[pl.program_id(0)] != 0)
    def _():
        y = jnp.dot(x_ref[...].astype(jnp.bfloat16),
                    we_ref[0].astype(jnp.bfloat16),
                    preferred_element_type=jnp.float32)
        y_ref[...] = y + be_ref[0]


def _gmm(x_pad, We, be, block_expert, block_valid):
    grid_spec = pltpu.PrefetchScalarGridSpec(
        num_scalar_prefetch=2,
        grid=(NB,),
        in_specs=[
            pl.BlockSpec((BM, D), lambda i, bmap, bval: (i, 0)),
            pl.BlockSpec((1, D, D), lambda i, bmap, bval: (bmap[i], 0, 0)),
            pl.BlockSpec((1, 1, D), lambda i, bmap, bval: (bmap[i], 0, 0)),
        ],
        out_specs=pl.BlockSpec((BM, D), lambda i, bmap, bval: (i, 0)),
    )
    return pl.pallas_call(
        _gmm_body,
        grid_spec=grid_spec,
        out_shape=jax.ShapeDtypeStruct((NPAD, D), jnp.float32),
    )(block_expert, block_valid, x_pad, We, be.reshape(E, 1, D))


_CCT = 16
_CNCH = (T // NW) // _CCT


def _combine_body(y_hbm, p0_hbm, p1_hbm, w0_hbm, w1_hbm, out_hbm,
                  i0_v, i1_v, w0_v, w1_v, r0_v, r1_v, o_v,
                  isem, gsem, wsem):
    per_w = T // NW
    wid = lax.axis_index("s") * 2 + lax.axis_index("c")
    tbase = wid * per_w

    pltpu.async_copy(p0_hbm.at[pl.ds(tbase, per_w)], i0_v, isem).wait()
    pltpu.async_copy(p1_hbm.at[pl.ds(tbase, per_w)], i1_v, isem).wait()

    def start_gather(c):
        b = c % 2
        s = pl.ds(c * _CCT, _CCT)
        hs = pl.ds(tbase + c * _CCT, _CCT)
        return (
            pltpu.async_copy(y_hbm.at[i0_v.at[s]], r0_v.at[b], gsem),
            pltpu.async_copy(y_hbm.at[i1_v.at[s]], r1_v.at[b], gsem),
            pltpu.async_copy(w0_hbm.at[hs], w0_v.at[b], gsem),
            pltpu.async_copy(w1_hbm.at[hs], w1_v.at[b], gsem),
        )

    gath = {0: start_gather(0), 1: start_gather(1)}
    stores = {}
    for c in range(_CNCH):
        b = c % 2
        for h in gath[c]:
            h.wait()
        if c >= 2:
            stores[c - 2].wait()

        def add_row(j, _):
            wa = w0_v[b, j, :]
            wb = w1_v[b, j, :]

            def add4(k, _):
                for u in range(4):
                    s = pl.ds((k * 4 + u) * 16, 16)
                    o_v[b, j, s] = r0_v[b, j, s] * wa + r1_v[b, j, s] * wb
                return 0
            lax.fori_loop(0, D // 64, add4, 0)
            return 0

        lax.fori_loop(0, _CCT, add_row, 0)
        stores[c] = pltpu.async_copy(
            o_v.at[b], out_hbm.at[pl.ds(tbase + c * _CCT, _CCT)], wsem)
        if c + 2 < _CNCH:
            gath[c + 2] = start_gather(c + 2)
    stores[_CNCH - 2].wait()
    stores[_CNCH - 1].wait()


def _combine(y_pad, pos0, pos1, w0e, w1e):
    per_w = T // NW
    return pl.kernel(
        _combine_body,
        out_type=jax.ShapeDtypeStruct((T, D), jnp.float32),
        mesh=plsc.VectorSubcoreMesh(**_SC_MESH),
        scratch_types=[
            pltpu.VMEM((per_w,), jnp.int32),
            pltpu.VMEM((per_w,), jnp.int32),
            pltpu.VMEM((2, _CCT, 16), jnp.float32),
            pltpu.VMEM((2, _CCT, 16), jnp.float32),
            pltpu.VMEM((2, _CCT, D), jnp.float32),
            pltpu.VMEM((2, _CCT, D), jnp.float32),
            pltpu.VMEM((2, _CCT, D), jnp.float32),
            pltpu.SemaphoreType.DMA,
            pltpu.SemaphoreType.DMA,
            pltpu.SemaphoreType.DMA,
        ],
    )(y_pad, pos0, pos1, w0e, w1e)


def kernel(inputs, Wg, bg, We, be):
    p0, p1, w0e, w1e, bmap, bval = _routing(inputs, Wg, bg)
    pos0 = p0.reshape(T)
    pos1 = p1.reshape(T)
    x_pad = _dispatch(inputs, pos0, pos1)
    y_pad = _gmm(x_pad, We, be, bmap.reshape(NB), bval.reshape(NB))
    return _combine(y_pad, pos0, pos1, w0e, w1e)


# speedup vs baseline: 2.5835x; 1.0408x over previous
import functools

import jax
import jax.numpy as jnp
from jax import lax
from jax.experimental import pallas as pl
from jax.experimental.pallas import tpu as pltpu
from jax.experimental.pallas import tpu_sc as plsc

T = 4096
D = 1024
E = 8
K = 2
N = T * K
BM = 256
NB = N // BM + E
NPAD = NB * BM

NW = 32
_SC_MESH = dict(core_axis_name="c", subcore_axis_name="s")


def _excl_cumsum(x):
    z = jnp.zeros_like(x)
    acc = x
    sh = 1
    while sh < T:
        acc = acc + jnp.concatenate([z[:sh], acc[: T - sh]], axis=0)
        sh *= 2
    return acc - x


def _route_sched_body(x_ref, wg_ref, bg_ref,
                      p0_ref, p1_ref, w0_ref, w1_ref, bmap_ref, bval_ref):
    x = x_ref[...]
    logits = jnp.dot(x, wg_ref[...], preferred_element_type=jnp.float32)
    logits = logits + bg_ref[...]
    e_iota = lax.broadcasted_iota(jnp.int32, logits.shape, 1)
    m1 = jnp.max(logits, axis=1, keepdims=True)
    i1 = jnp.min(jnp.where(logits == m1, e_iota, E), axis=1, keepdims=True)
    masked = jnp.where(e_iota == i1, -jnp.inf, logits)
    m2 = jnp.max(masked, axis=1, keepdims=True)
    i2 = jnp.min(jnp.where(masked == m2, e_iota, E), axis=1, keepdims=True)
    z = jnp.exp(m2 - m1)
    w1 = 1.0 / (1.0 + z)
    w0_ref[...] = jnp.broadcast_to(w1, (T, 16))
    w1_ref[...] = jnp.broadcast_to(1.0 - w1, (T, 16))

    oh1 = (e_iota == i1).astype(jnp.int32)
    oh2 = (e_iota == i2).astype(jnp.int32)
    oh = oh1 + oh2
    r = _excl_cumsum(oh)
    counts = r[T - 1:T, :] + oh[T - 1:T, :]
    blocks_per = (counts + BM - 1) // BM
    ri = lax.broadcasted_iota(jnp.int32, (E, E), 0)
    ci = lax.broadcasted_iota(jnp.int32, (E, E), 1)
    tri = (ri <= ci).astype(jnp.float32)
    cumblocks = jnp.dot(blocks_per.astype(jnp.float32), tri,
                        preferred_element_type=jnp.float32).astype(jnp.int32)
    padded_start = (cumblocks - blocks_per) * BM

    dest1 = jnp.sum(oh1 * (padded_start + r), axis=1, keepdims=True)
    dest2 = jnp.sum(oh2 * (padded_start + r + oh1), axis=1, keepdims=True)
    p0_ref[...] = dest1.reshape(NW, T // NW)
    p1_ref[...] = dest2.reshape(NW, T // NW)

    b_iota = lax.broadcasted_iota(jnp.int32, (NB, E), 0)
    bmap = jnp.sum((jnp.broadcast_to(cumblocks, (NB, E)) <= b_iota)
                   .astype(jnp.int32), axis=1, keepdims=True)
    bmap_ref[...] = jnp.minimum(bmap, E - 1)
    bval_ref[...] = (b_iota[:, :1] < cumblocks[0, E - 1]).astype(jnp.int32)


def _routing(inputs, Wg, bg):
    return pl.pallas_call(
        _route_sched_body,
        grid=(1,),
        in_specs=[
            pl.BlockSpec((T, D), lambda i: (0, 0)),
            pl.BlockSpec((D, E), lambda i: (0, 0)),
            pl.BlockSpec((1, E), lambda i: (0, 0)),
        ],
        out_specs=[
            pl.BlockSpec((NW, T // NW), lambda i: (0, 0)),
            pl.BlockSpec((NW, T // NW), lambda i: (0, 0)),
            pl.BlockSpec((T, 16), lambda i: (0, 0)),
            pl.BlockSpec((T, 16), lambda i: (0, 0)),
            pl.BlockSpec((NB, 1), lambda i: (0, 0)),
            pl.BlockSpec((NB, 1), lambda i: (0, 0)),
        ],
        out_shape=[
            jax.ShapeDtypeStruct((NW, T // NW), jnp.int32),
            jax.ShapeDtypeStruct((NW, T // NW), jnp.int32),
            jax.ShapeDtypeStruct((T, 16), jnp.float32),
            jax.ShapeDtypeStruct((T, 16), jnp.float32),
            jax.ShapeDtypeStruct((NB, 1), jnp.int32),
            jax.ShapeDtypeStruct((NB, 1), jnp.int32),
        ],
    )(inputs, Wg, bg.reshape(1, E))


_DCT = 32
_DNCH = (T // NW) // _DCT


def _dispatch_body(x_hbm, p0_hbm, p1_hbm, out_hbm, i0_v, i1_v, xin_v,
                   lsem, ssem):
    per_w = T // NW
    wid = lax.axis_index("s") * 2 + lax.axis_index("c")
    tbase = wid * per_w

    def start_load(c):
        b = c % 2
        base = tbase + c * _DCT
        return (
            pltpu.async_copy(p0_hbm.at[wid, pl.ds(c * _DCT, _DCT)],
                             i0_v.at[b], lsem),
            pltpu.async_copy(p1_hbm.at[wid, pl.ds(c * _DCT, _DCT)],
                             i1_v.at[b], lsem),
            pltpu.async_copy(x_hbm.at[pl.ds(base, _DCT)], xin_v.at[b], lsem),
        )

    loads = {0: start_load(0), 1: start_load(1)}
    scats = {}
    for c in range(_DNCH):
        b = c % 2
        for h in loads[c]:
            h.wait()
        scats[c] = (
            pltpu.async_copy(xin_v.at[b], out_hbm.at[i0_v.at[b]], ssem),
            pltpu.async_copy(xin_v.at[b], out_hbm.at[i1_v.at[b]], ssem),
        )
        if c + 2 < _DNCH:
            for h in scats[c]:
                h.wait()
            loads[c + 2] = start_load(c + 2)
    for c in (_DNCH - 2, _DNCH - 1):
        for h in scats[c]:
            h.wait()


def _dispatch(inputs, pos0, pos1):
    return pl.kernel(
        _dispatch_body,
        out_type=jax.ShapeDtypeStruct((NPAD, D), jnp.float32),
        mesh=plsc.VectorSubcoreMesh(**_SC_MESH),
        scratch_types=[
            pltpu.VMEM((2, _DCT), jnp.int32),
            pltpu.VMEM((2, _DCT), jnp.int32),
            pltpu.VMEM((2, _DCT, D), jnp.float32),
            pltpu.SemaphoreType.DMA,
            pltpu.SemaphoreType.DMA,
        ],
    )(inputs, pos0, pos1)


def _gmm_body(bmap_ref, bval_ref, x_ref, we_ref, be_ref, y_ref):
    del bmap_ref

    @pl.when(bval_ref[pl.program_id(0)] != 0)
    def _():
        y = jnp.dot(x_ref[...].astype(jnp.bfloat16),
                    we_ref[0].astype(jnp.bfloat16),
                    preferred_element_type=jnp.float32)
        y_ref[...] = y + be_ref[0]


def _gmm(x_pad, We, be, block_expert, block_valid):
    grid_spec = pltpu.PrefetchScalarGridSpec(
        num_scalar_prefetch=2,
        grid=(NB,),
        in_specs=[
            pl.BlockSpec((BM, D), lambda i, bmap, bval: (i, 0)),
            pl.BlockSpec((1, D, D), lambda i, bmap, bval: (bmap[i], 0, 0)),
            pl.BlockSpec((1, 1, D), lambda i, bmap, bval: (bmap[i], 0, 0)),
        ],
        out_specs=pl.BlockSpec((BM, D), lambda i, bmap, bval: (i, 0)),
    )
    return pl.pallas_call(
        _gmm_body,
        grid_spec=grid_spec,
        out_shape=jax.ShapeDtypeStruct((NPAD, D), jnp.float32),
    )(block_expert, block_valid, x_pad, We, be.reshape(E, 1, D))


_CCT = 16
_CNCH = (T // NW) // _CCT


def _combine_body(y_hbm, p0_hbm, p1_hbm, w0_hbm, w1_hbm, out_hbm,
                  i0_v, i1_v, w0_v, w1_v, r0_v, r1_v, o_v,
                  isem, gsem, wsem):
    per_w = T // NW
    wid = lax.axis_index("s") * 2 + lax.axis_index("c")
    tbase = wid * per_w

    pltpu.async_copy(p0_hbm.at[wid], i0_v, isem).wait()
    pltpu.async_copy(p1_hbm.at[wid], i1_v, isem).wait()

    def start_gather(c):
        b = c % 2
        s = pl.ds(c * _CCT, _CCT)
        hs = pl.ds(tbase + c * _CCT, _CCT)
        return (
            pltpu.async_copy(y_hbm.at[i0_v.at[s]], r0_v.at[b], gsem),
            pltpu.async_copy(y_hbm.at[i1_v.at[s]], r1_v.at[b], gsem),
            pltpu.async_copy(w0_hbm.at[hs], w0_v.at[b], gsem),
            pltpu.async_copy(w1_hbm.at[hs], w1_v.at[b], gsem),
        )

    gath = {0: start_gather(0), 1: start_gather(1)}
    stores = {}
    for c in range(_CNCH):
        b = c % 2
        for h in gath[c]:
            h.wait()
        if c >= 2:
            stores[c - 2].wait()

        def add_row(j, _):
            wa = w0_v[b, j, :]
            wb = w1_v[b, j, :]

            def add4(k, _):
                for u in range(4):
                    s = pl.ds((k * 4 + u) * 16, 16)
                    o_v[b, j, s] = r0_v[b, j, s] * wa + r1_v[b, j, s] * wb
                return 0
            lax.fori_loop(0, D // 64, add4, 0)
            return 0

        lax.fori_loop(0, _CCT, add_row, 0)
        stores[c] = pltpu.async_copy(
            o_v.at[b], out_hbm.at[pl.ds(tbase + c * _CCT, _CCT)], wsem)
        if c + 2 < _CNCH:
            gath[c + 2] = start_gather(c + 2)
    stores[_CNCH - 2].wait()
    stores[_CNCH - 1].wait()


def _combine(y_pad, pos0, pos1, w0e, w1e):
    per_w = T // NW
    return pl.kernel(
        _combine_body,
        out_type=jax.ShapeDtypeStruct((T, D), jnp.float32),
        mesh=plsc.VectorSubcoreMesh(**_SC_MESH),
        scratch_types=[
            pltpu.VMEM((per_w,), jnp.int32),
            pltpu.VMEM((per_w,), jnp.int32),
            pltpu.VMEM((2, _CCT, 16), jnp.float32),
            pltpu.VMEM((2, _CCT, 16), jnp.float32),
            pltpu.VMEM((2, _CCT, D), jnp.float32),
            pltpu.VMEM((2, _CCT, D), jnp.float32),
            pltpu.VMEM((2, _CCT, D), jnp.float32),
            pltpu.SemaphoreType.DMA,
            pltpu.SemaphoreType.DMA,
            pltpu.SemaphoreType.DMA,
        ],
    )(y_pad, pos0, pos1, w0e, w1e)


def kernel(inputs, Wg, bg, We, be):
    p0, p1, w0e, w1e, bmap, bval = _routing(inputs, Wg, bg)
    x_pad = _dispatch(inputs, p0, p1)
    y_pad = _gmm(x_pad, We, be, bmap.reshape(NB), bval.reshape(NB))
    return _combine(y_pad, p0, p1, w0e, w1e)
